# baseline (speedup 1.0000x reference)
import functools

import jax
import jax.numpy as jnp
from jax import lax
from jax.experimental import pallas as pl
from jax.experimental.pallas import tpu as pltpu
from jax.experimental.pallas import tpu_sc as plsc

N = 10000
E = 320000
D = 128
DQ = 32

NC = 2
NS = 16
NW = NC * NS
EPW = E // NW
C = 80
NCH = EPW // C
CS = 80
NCHS = EPW // CS
RPT = N // NS

_f32 = jnp.float32


def _k1_body(x_ref, pos_ref, wma, wmb, wu1x, wp1,
             a_ref, b_ref, x1_ref, p_ref):
    x = x_ref[...]
    a_ref[...] = jnp.dot(x, wma[...], preferred_element_type=_f32)
    b_ref[...] = jnp.dot(x, wmb[...], preferred_element_type=_f32)
    x1_ref[...] = jnp.dot(x, wu1x[...], preferred_element_type=_f32)
    p_ref[...] = jnp.dot(pos_ref[...], wp1[...], preferred_element_type=_f32)


def _k2_body(h_ref, w2c4_ref, b2c4_ref, p_ref):
    h4 = h_ref[...]
    q = jnp.dot(h4, w2c4_ref[...], preferred_element_type=_f32) + b2c4_ref[...]
    p_ref[...] = q.reshape(q.shape[0] * 4, D)


def _k4_body(x1_ref, s0_ref, s1_ref, d0_ref, d1_ref,
             wm2, bm2, wu1g, bu1, wu2, bu2, out_ref):
    s = s0_ref[...] + s1_ref[...]
    deg = (d0_ref[...] + d1_ref[...])[:, 0:1]
    agg = jnp.dot(s, wm2[...], preferred_element_type=_f32) + deg * bm2[...]
    u = jnp.maximum(
        x1_ref[...] + jnp.dot(agg, wu1g[...], preferred_element_type=_f32)
        + bu1[...], 0.0)
    out_ref[...] = jnp.dot(u, wu2[...], preferred_element_type=_f32) + bu2[...]


_MESH = plsc.VectorSubcoreMesh(core_axis_name="c", subcore_axis_name="s",
                               num_cores=NC, num_subcores=NS)


@functools.partial(
    pl.kernel,
    out_type=(jax.ShapeDtypeStruct((E // 4, D), _f32),
              jax.ShapeDtypeStruct((NC, N, 16), _f32)),
    mesh=_MESH,
    compiler_params=pltpu.CompilerParams(use_tc_tiling_on_sc=False),
    scratch_types=[
        pltpu.VMEM((C,), jnp.int32),
        pltpu.VMEM((C,), jnp.int32),
        pltpu.VMEM((C,), jnp.int32),
        pltpu.VMEM((C,), jnp.int32),
        pltpu.VMEM((C, DQ), _f32),
        pltpu.VMEM((C, DQ), _f32),
        pltpu.VMEM((C, DQ), _f32),
        pltpu.VMEM((C, DQ), _f32),
        pltpu.VMEM((C // 4, D), _f32),
        pltpu.VMEM((C // 4, D), _f32),
        pltpu.VMEM((DQ,), _f32),
        pltpu.VMEM((C, 16), _f32),
        pltpu.VMEM((RPT, 16), _f32),
        pltpu.VMEM_SHARED((N, 16), _f32),
        pltpu.SemaphoreType.DMA,
        pltpu.SemaphoreType.DMA,
        pltpu.SemaphoreType.DMA,
        pltpu.SemaphoreType.DMA,
    ],
)
def _sc_h(p_hbm, row_hbm, col_hbm, bp1_hbm, h_hbm, d_out,
          ridx0, ridx1, cidx0, cidx1, pr0, pr1, pc0, pc1, hv0, hv1,
          bp1v, ones, zd, d_sh, sem0, sem1, hsem0, hsem1):
    cid = lax.axis_index("c")
    sid = lax.axis_index("s")
    w = cid * NS + sid
    pltpu.sync_copy(bp1_hbm, bp1v)
    ridxs, cidxs = (ridx0, ridx1), (cidx0, cidx1)
    prs, pcs, hvs = (pr0, pr1), (pc0, pc1), (hv0, hv1)
    sems, hsems = (sem0, sem1), (hsem0, hsem1)

    zero16 = jnp.zeros((16,), _f32)

    def zd_init(i, c2):
        zd[i, pl.ds(0, 16)] = zero16
        return c2

    lax.fori_loop(0, RPT, zd_init, 0)

    def ones_init(i, c2):
        ones[i, pl.ds(0, 16)] = jnp.ones((16,), _f32)
        return c2

    lax.fori_loop(0, C, ones_init, 0)

    pltpu.sync_copy(zd, d_sh.at[pl.ds(sid * RPT, RPT)])
    plsc.subcore_barrier()

    b16 = (bp1v[pl.ds(0, 16)], bp1v[pl.ds(16, 16)])

    def hdst(ci):
        return h_hbm.at[pl.ds(w * (EPW // 4) + ci * (C // 4), C // 4)]

    def fire(ci, buf):
        base = w * EPW + ci * C
        pltpu.sync_copy(row_hbm.at[pl.ds(base, C)], ridxs[buf])
        pltpu.sync_copy(col_hbm.at[pl.ds(base, C)], cidxs[buf])
        pltpu.async_copy(p_hbm.at[ridxs[buf]], prs[buf], sems[buf])
        pltpu.async_copy(p_hbm.at[cidxs[buf]], pcs[buf], sems[buf])

    def consume(ci, buf):
        pr, pc, hv = prs[buf], pcs[buf], hvs[buf]
        pltpu.make_async_copy(p_hbm.at[ridxs[buf]], pr, sems[buf]).wait()
        pltpu.make_async_copy(p_hbm.at[cidxs[buf]], pc, sems[buf]).wait()

        @pl.when(ci >= 2)
        def _():
            pltpu.make_async_copy(hv, hdst(ci - 2), hsems[buf]).wait()

        def body(i, c2):
            for k in range(4):
                for jj in range(DQ // 16):
                    sl = pl.ds(jj * 16, 16)
                    v = pr[4 * i + k, sl] - pc[4 * i + k, sl] + b16[jj]
                    hv[i, pl.ds(k * DQ + jj * 16, 16)] = jnp.maximum(v, 0.0)
            return c2

        lax.fori_loop(0, C // 4, body, 0)
        pltpu.async_copy(hv, hdst(ci), hsems[buf])
        pltpu.sync_copy(ones, d_sh.at[cidxs[buf]], add=True)

    fire(0, 0)

    def pair(k, carry):
        ci = k * 2
        fire(ci + 1, 1)
        consume(ci, 0)

        @pl.when(ci + 2 < NCH)
        def _():
            fire(ci + 2, 0)

        consume(ci + 1, 1)
        return carry

    lax.fori_loop(0, NCH // 2, pair, 0)
    consume(jnp.int32(NCH - 1), 0)
    pltpu.make_async_copy(hvs[0], hdst(NCH - 1), hsems[0]).wait()
    pltpu.make_async_copy(hvs[1], hdst(NCH - 2), hsems[1]).wait()
    plsc.subcore_barrier()
    sl = pl.ds(sid * RPT, RPT)
    pltpu.sync_copy(d_sh.at[sl], d_out.at[cid, sl])


@functools.partial(
    pl.kernel,
    out_type=jax.ShapeDtypeStruct((NC, N, D), _f32),
    mesh=_MESH,
    compiler_params=pltpu.CompilerParams(use_tc_tiling_on_sc=False),
    scratch_types=[
        pltpu.VMEM((CS,), jnp.int32),
        pltpu.VMEM((CS,), jnp.int32),
        pltpu.VMEM((CS,), jnp.int32),
        pltpu.VMEM((CS,), jnp.int32),
        pltpu.VMEM((CS,), jnp.int32),
        pltpu.VMEM((CS,), jnp.int32),
        pltpu.VMEM((CS,), jnp.int32),
        pltpu.VMEM((CS,), jnp.int32),
        pltpu.VMEM((CS, D), _f32),
        pltpu.VMEM((CS, D), _f32),
        pltpu.VMEM((CS, D), _f32),
        pltpu.VMEM((CS, D), _f32),
        pltpu.VMEM_SHARED((N, D), _f32),
        pltpu.SemaphoreType.DMA,
        pltpu.SemaphoreType.DMA,
        pltpu.SemaphoreType.DMA,
        pltpu.SemaphoreType.DMA,
        pltpu.SemaphoreType.DMA,
        pltpu.SemaphoreType.DMA,
    ],
)
def _sc_scatter(a_hbm, b_hbm, p_hbm, row_hbm, col_hbm, s_out,
                ridx0, ridx1, cidx0, cidx1, scidx0, scidx1, pidx0, pidx1,
                av0, av1, mv0, mv1, s_sh,
                asem0, asem1, gsem0, gsem1, ssem0, ssem1):
    cid = lax.axis_index("c")
    sid = lax.axis_index("s")
    w = cid * NS + sid
    ridxs, cidxs = (ridx0, ridx1), (cidx0, cidx1)
    scidxs, pidxs = (scidx0, scidx1), (pidx0, pidx1)
    avs, mvs = (av0, av1), (mv0, mv1)
    asems, gsems, ssems = (asem0, asem1), (gsem0, gsem1), (ssem0, ssem1)

    zero16 = jnp.zeros((16,), _f32)
    iota16 = lax.iota(jnp.int32, 16)

    def z_init(i, c2):
        for j in range(D // 16):
            av0[i, pl.ds(j * 16, 16)] = zero16
        return c2

    lax.fori_loop(0, CS, z_init, 0)

    for k in range(RPT // CS):
        pltpu.sync_copy(av0, s_sh.at[pl.ds(sid * RPT + k * CS, CS)])
    rem = RPT - (RPT // CS) * CS
    if rem:
        pltpu.sync_copy(av0.at[pl.ds(0, rem)],
                        s_sh.at[pl.ds(sid * RPT + RPT - rem, rem)])
    plsc.subcore_barrier()

    def fire(ci, buf):
        base = w * EPW + ci * CS
        pltpu.sync_copy(row_hbm.at[pl.ds(base, CS)], ridxs[buf])
        pltpu.sync_copy(col_hbm.at[pl.ds(base, CS)], cidxs[buf])
        pltpu.async_copy(a_hbm.at[ridxs[buf]], avs[buf], asems[buf])
        for k in range(CS // 16):
            pidxs[buf][pl.ds(k * 16, 16)] = iota16 + (base + k * 16)

    def gfire(ci, buf):
        pltpu.make_async_copy(a_hbm.at[ridxs[buf]], avs[buf],
                              asems[buf]).wait()
        pltpu.async_copy(b_hbm.at[cidxs[buf]], avs[buf], gsems[buf],
                         add=True)
        pltpu.async_copy(p_hbm.at[pidxs[buf]], avs[buf], gsems[buf],
                         add=True)

    def consume(ci, buf):
        av, mv = avs[buf], mvs[buf]
        pltpu.make_async_copy(b_hbm.at[cidxs[buf]], av, gsems[buf]).wait()
        pltpu.make_async_copy(p_hbm.at[pidxs[buf]], av, gsems[buf]).wait()

        @pl.when(ci >= 2)
        def _():
            pltpu.make_async_copy(mv, s_sh.at[scidxs[buf]],
                                  ssems[buf]).wait()

        def body(i, c2):
            for j in range(D // 16):
                sl = pl.ds(j * 16, 16)
                mv[i, sl] = jnp.maximum(av[i, sl], 0.0)
            return c2

        lax.fori_loop(0, CS, body, 0)
        for k in range(CS // 16):
            sl = pl.ds(k * 16, 16)
            scidxs[buf][sl] = cidxs[buf][sl]
        pltpu.async_copy(mv, s_sh.at[scidxs[buf]], ssems[buf], add=True)

    fire(0, 0)
    fire(1, 1)
    gfire(0, 0)

    def pair(k, carry):
        ci = k * 2
        consume(ci, 0)

        @pl.when(ci + 2 < NCHS)
        def _():
            fire(ci + 2, 0)

        gfire(ci + 1, 1)
        consume(ci + 1, 1)

        @pl.when(ci + 3 < NCHS)
        def _():
            fire(ci + 3, 1)

        @pl.when(ci + 2 < NCHS)
        def _():
            gfire(ci + 2, 0)

        return carry

    lax.fori_loop(0, NCHS // 2, pair, 0)
    if NCHS % 2:
        consume(jnp.int32(NCHS - 1), 0)
    pltpu.make_async_copy(mvs[0], s_sh.at[scidxs[0]], ssems[0]).wait()
    pltpu.make_async_copy(mvs[1], s_sh.at[scidxs[1]], ssems[1]).wait()
    plsc.subcore_barrier()

    sl = pl.ds(sid * RPT, RPT)
    pltpu.sync_copy(s_sh.at[sl], s_out.at[cid, sl])


def kernel(x, pos, edge_index, Wp1, bp1, Wp2, bp2, Wm1, bm1, Wm2, bm2,
           Wu1, bu1, Wu2, bu2):
    row = edge_index[0]
    col = edge_index[1]
    pos8 = jnp.pad(pos, ((0, 0), (0, 5)))
    wp18 = jnp.pad(Wp1, ((0, 5), (0, 0)))
    W2c = Wp2 @ Wm1[2 * D:]
    b2c = (bp2 @ Wm1[2 * D:] + bm1).reshape(1, D)
    W2c4 = jnp.zeros((D, 4 * D), _f32)
    for k in range(4):
        W2c4 = W2c4.at[k * DQ:(k + 1) * DQ, k * D:(k + 1) * D].set(W2c)
    b2c4 = jnp.tile(b2c, (1, 4))

    RB = 1000
    nb = N // RB
    k1 = pl.pallas_call(
        _k1_body,
        grid=(nb,),
        in_specs=[
            pl.BlockSpec((RB, D), lambda i: (i, 0)),
            pl.BlockSpec((RB, 8), lambda i: (i, 0)),
            pl.BlockSpec((D, D), lambda i: (0, 0)),
            pl.BlockSpec((D, D), lambda i: (0, 0)),
            pl.BlockSpec((D, D), lambda i: (0, 0)),
            pl.BlockSpec((8, DQ), lambda i: (0, 0)),
        ],
        out_specs=[
            pl.BlockSpec((RB, D), lambda i: (i, 0)),
            pl.BlockSpec((RB, D), lambda i: (i, 0)),
            pl.BlockSpec((RB, D), lambda i: (i, 0)),
            pl.BlockSpec((RB, DQ), lambda i: (i, 0)),
        ],
        out_shape=[
            jax.ShapeDtypeStruct((N, D), _f32),
            jax.ShapeDtypeStruct((N, D), _f32),
            jax.ShapeDtypeStruct((N, D), _f32),
            jax.ShapeDtypeStruct((N, DQ), _f32),
        ],
    )
    A, B, X1, P = k1(x, pos8, Wm1[:D], Wm1[D:2 * D], Wu1[:D], wp18)

    h, Dg = _sc_h(P, row, col, bp1)

    RE4 = 1000
    k2 = pl.pallas_call(
        _k2_body,
        grid=(E // 4 // RE4,),
        in_specs=[
            pl.BlockSpec((RE4, D), lambda i: (i, 0)),
            pl.BlockSpec((D, 4 * D), lambda i: (0, 0)),
            pl.BlockSpec((1, 4 * D), lambda i: (0, 0)),
        ],
        out_specs=pl.BlockSpec((4 * RE4, D), lambda i: (i, 0)),
        out_shape=jax.ShapeDtypeStruct((E, D), _f32),
    )
    p = k2(h, W2c4, b2c4)

    S = _sc_scatter(A, B, p, row, col)

    k4 = pl.pallas_call(
        _k4_body,
        grid=(nb,),
        in_specs=[
            pl.BlockSpec((RB, D), lambda i: (i, 0)),
            pl.BlockSpec((RB, D), lambda i: (i, 0)),
            pl.BlockSpec((RB, D), lambda i: (i, 0)),
            pl.BlockSpec((RB, 16), lambda i: (i, 0)),
            pl.BlockSpec((RB, 16), lambda i: (i, 0)),
            pl.BlockSpec((D, D), lambda i: (0, 0)),
            pl.BlockSpec((1, D), lambda i: (0, 0)),
            pl.BlockSpec((D, D), lambda i: (0, 0)),
            pl.BlockSpec((1, D), lambda i: (0, 0)),
            pl.BlockSpec((D, D), lambda i: (0, 0)),
            pl.BlockSpec((1, D), lambda i: (0, 0)),
        ],
        out_specs=pl.BlockSpec((RB, D), lambda i: (i, 0)),
        out_shape=jax.ShapeDtypeStruct((N, D), _f32),
    )
    out = k4(X1, S[0], S[1], Dg[0], Dg[1], Wm2, bm2.reshape(1, D),
             Wu1[D:], bu1.reshape(1, D), Wu2, bu2.reshape(1, D))
    return out


# speedup vs baseline: 1.1647x; 1.1647x over previous
import functools

import jax
import jax.numpy as jnp
from jax import lax
from jax.experimental import pallas as pl
from jax.experimental.pallas import tpu as pltpu
from jax.experimental.pallas import tpu_sc as plsc

N = 10000
E = 320000
D = 128
DQ = 32

NC = 2
NS = 16
NW = NC * NS
EPW = E // NW
C = 80
NCH = EPW // C
CS = 80
NCHS = EPW // CS
RPT = N // NS

_f32 = jnp.float32


def _k1_body(x_ref, pos_ref, wma, wmb, wu1x, wp1,
             a_ref, b_ref, x1_ref, p_ref):
    x = x_ref[...]
    a_ref[...] = jnp.dot(x, wma[...], preferred_element_type=_f32)
    b_ref[...] = jnp.dot(x, wmb[...], preferred_element_type=_f32)
    x1_ref[...] = jnp.dot(x, wu1x[...], preferred_element_type=_f32)
    p_ref[...] = jnp.dot(pos_ref[...], wp1[...], preferred_element_type=_f32)


def _k2_body(h_ref, w2c4_ref, b2c4_ref, p_ref):
    h4 = h_ref[...]
    q = jnp.dot(h4, w2c4_ref[...], preferred_element_type=_f32) + b2c4_ref[...]
    p_ref[...] = q.reshape(q.shape[0] * 4, D)


def _k4_body(x1_ref, s0_ref, s1_ref, d0_ref, d1_ref,
             wm2, bm2, wu1g, bu1, wu2, bu2, out_ref):
    s = s0_ref[...] + s1_ref[...]
    deg = (d0_ref[...] + d1_ref[...])[:, 0:1]
    agg = jnp.dot(s, wm2[...], preferred_element_type=_f32) + deg * bm2[...]
    u = jnp.maximum(
        x1_ref[...] + jnp.dot(agg, wu1g[...], preferred_element_type=_f32)
        + bu1[...], 0.0)
    out_ref[...] = jnp.dot(u, wu2[...], preferred_element_type=_f32) + bu2[...]


_MESH = plsc.VectorSubcoreMesh(core_axis_name="c", subcore_axis_name="s",
                               num_cores=NC, num_subcores=NS)


@functools.partial(
    pl.kernel,
    out_type=(jax.ShapeDtypeStruct((E // 4, D), _f32),
              jax.ShapeDtypeStruct((NC, N, 16), _f32)),
    mesh=_MESH,
    compiler_params=pltpu.CompilerParams(use_tc_tiling_on_sc=False),
    scratch_types=[
        pltpu.VMEM((C,), jnp.int32),
        pltpu.VMEM((C,), jnp.int32),
        pltpu.VMEM((C,), jnp.int32),
        pltpu.VMEM((C,), jnp.int32),
        pltpu.VMEM((C, DQ), _f32),
        pltpu.VMEM((C, DQ), _f32),
        pltpu.VMEM((C, DQ), _f32),
        pltpu.VMEM((C, DQ), _f32),
        pltpu.VMEM((C // 4, D), _f32),
        pltpu.VMEM((C // 4, D), _f32),
        pltpu.VMEM((DQ,), _f32),
        pltpu.VMEM((C, 16), _f32),
        pltpu.VMEM((RPT, 16), _f32),
        pltpu.VMEM_SHARED((N, 16), _f32),
        pltpu.SemaphoreType.DMA,
        pltpu.SemaphoreType.DMA,
        pltpu.SemaphoreType.DMA,
        pltpu.SemaphoreType.DMA,
    ],
)
def _sc_h(p_hbm, row_hbm, col_hbm, bp1_hbm, h_hbm, d_out,
          ridx0, ridx1, cidx0, cidx1, pr0, pr1, pc0, pc1, hv0, hv1,
          bp1v, ones, zd, d_sh, sem0, sem1, hsem0, hsem1):
    cid = lax.axis_index("c")
    sid = lax.axis_index("s")
    w = cid * NS + sid
    pltpu.sync_copy(bp1_hbm, bp1v)
    ridxs, cidxs = (ridx0, ridx1), (cidx0, cidx1)
    prs, pcs, hvs = (pr0, pr1), (pc0, pc1), (hv0, hv1)
    sems, hsems = (sem0, sem1), (hsem0, hsem1)

    zero16 = jnp.zeros((16,), _f32)

    def zd_init(i, c2):
        zd[i, pl.ds(0, 16)] = zero16
        return c2

    lax.fori_loop(0, RPT, zd_init, 0)

    def ones_init(i, c2):
        ones[i, pl.ds(0, 16)] = jnp.ones((16,), _f32)
        return c2

    lax.fori_loop(0, C, ones_init, 0)

    pltpu.sync_copy(zd, d_sh.at[pl.ds(sid * RPT, RPT)])
    plsc.subcore_barrier()

    b16 = (bp1v[pl.ds(0, 16)], bp1v[pl.ds(16, 16)])

    def hdst(ci):
        return h_hbm.at[pl.ds(w * (EPW // 4) + ci * (C // 4), C // 4)]

    def fire(ci, buf):
        base = w * EPW + ci * C
        pltpu.sync_copy(row_hbm.at[pl.ds(base, C)], ridxs[buf])
        pltpu.sync_copy(col_hbm.at[pl.ds(base, C)], cidxs[buf])
        pltpu.async_copy(p_hbm.at[ridxs[buf]], prs[buf], sems[buf])
        pltpu.async_copy(p_hbm.at[cidxs[buf]], pcs[buf], sems[buf])

    def consume(ci, buf):
        pr, pc, hv = prs[buf], pcs[buf], hvs[buf]
        pltpu.make_async_copy(p_hbm.at[ridxs[buf]], pr, sems[buf]).wait()
        pltpu.make_async_copy(p_hbm.at[cidxs[buf]], pc, sems[buf]).wait()

        @pl.when(ci >= 2)
        def _():
            pltpu.make_async_copy(hv, hdst(ci - 2), hsems[buf]).wait()

        def body(i, c2):
            for k in range(4):
                for jj in range(DQ // 16):
                    sl = pl.ds(jj * 16, 16)
                    v = pr[4 * i + k, sl] - pc[4 * i + k, sl] + b16[jj]
                    hv[i, pl.ds(k * DQ + jj * 16, 16)] = jnp.maximum(v, 0.0)
            return c2

        lax.fori_loop(0, C // 4, body, 0)
        pltpu.async_copy(hv, hdst(ci), hsems[buf])
        pltpu.sync_copy(ones, d_sh.at[cidxs[buf]], add=True)

    fire(0, 0)

    def pair(k, carry):
        ci = k * 2
        fire(ci + 1, 1)
        consume(ci, 0)

        @pl.when(ci + 2 < NCH)
        def _():
            fire(ci + 2, 0)

        consume(ci + 1, 1)
        return carry

    lax.fori_loop(0, NCH // 2, pair, 0)
    consume(jnp.int32(NCH - 1), 0)
    pltpu.make_async_copy(hvs[0], hdst(NCH - 1), hsems[0]).wait()
    pltpu.make_async_copy(hvs[1], hdst(NCH - 2), hsems[1]).wait()
    plsc.subcore_barrier()
    sl = pl.ds(sid * RPT, RPT)
    pltpu.sync_copy(d_sh.at[sl], d_out.at[cid, sl])


@functools.partial(
    pl.kernel,
    out_type=jax.ShapeDtypeStruct((NC, N, D), _f32),
    mesh=_MESH,
    compiler_params=pltpu.CompilerParams(use_tc_tiling_on_sc=False),
    scratch_types=[
        pltpu.VMEM((CS,), jnp.int32),
        pltpu.VMEM((CS,), jnp.int32),
        pltpu.VMEM((CS,), jnp.int32),
        pltpu.VMEM((CS,), jnp.int32),
        pltpu.VMEM((CS,), jnp.int32),
        pltpu.VMEM((CS,), jnp.int32),
        pltpu.VMEM((CS,), jnp.int32),
        pltpu.VMEM((CS,), jnp.int32),
        pltpu.VMEM((CS, D), _f32),
        pltpu.VMEM((CS, D), _f32),
        pltpu.VMEM((CS, D), _f32),
        pltpu.VMEM((CS, D), _f32),
        pltpu.VMEM_SHARED((N, D), _f32),
        pltpu.SemaphoreType.DMA,
        pltpu.SemaphoreType.DMA,
        pltpu.SemaphoreType.DMA,
        pltpu.SemaphoreType.DMA,
        pltpu.SemaphoreType.DMA,
        pltpu.SemaphoreType.DMA,
    ],
)
def _sc_scatter(a_hbm, b_hbm, p_hbm, row_hbm, col_hbm, s_out,
                ridx0, ridx1, cidx0, cidx1, scidx0, scidx1, pidx0, pidx1,
                av0, av1, mv0, mv1, s_sh,
                asem0, asem1, gsem0, gsem1, ssem0, ssem1):
    cid = lax.axis_index("c")
    sid = lax.axis_index("s")
    w = cid * NS + sid
    ridxs, cidxs = (ridx0, ridx1), (cidx0, cidx1)
    scidxs, pidxs = (scidx0, scidx1), (pidx0, pidx1)
    avs, mvs = (av0, av1), (mv0, mv1)
    asems, gsems, ssems = (asem0, asem1), (gsem0, gsem1), (ssem0, ssem1)

    zero16 = jnp.zeros((16,), _f32)
    iota16 = lax.iota(jnp.int32, 16)

    def z_init(i, c2):
        for j in range(D // 16):
            av0[i, pl.ds(j * 16, 16)] = zero16
        return c2

    lax.fori_loop(0, CS, z_init, 0)

    for k in range(RPT // CS):
        pltpu.sync_copy(av0, s_sh.at[pl.ds(sid * RPT + k * CS, CS)])
    rem = RPT - (RPT // CS) * CS
    if rem:
        pltpu.sync_copy(av0.at[pl.ds(0, rem)],
                        s_sh.at[pl.ds(sid * RPT + RPT - rem, rem)])
    plsc.subcore_barrier()

    def fire(ci, buf):
        base = w * EPW + ci * CS
        pltpu.sync_copy(row_hbm.at[pl.ds(base, CS)], ridxs[buf])
        pltpu.sync_copy(col_hbm.at[pl.ds(base, CS)], cidxs[buf])
        pltpu.async_copy(a_hbm.at[ridxs[buf]], avs[buf], asems[buf])
        for k in range(CS // 16):
            pidxs[buf][pl.ds(k * 16, 16)] = iota16 + (base + k * 16)

    def gfire(ci, buf):
        pltpu.make_async_copy(a_hbm.at[ridxs[buf]], avs[buf],
                              asems[buf]).wait()
        pltpu.async_copy(b_hbm.at[cidxs[buf]], avs[buf], gsems[buf],
                         add=True)
        pltpu.async_copy(p_hbm.at[pidxs[buf]], avs[buf], gsems[buf],
                         add=True)

    def consume(ci, buf):
        av, mv = avs[buf], mvs[buf]
        pltpu.make_async_copy(b_hbm.at[cidxs[buf]], av, gsems[buf]).wait()
        pltpu.make_async_copy(p_hbm.at[pidxs[buf]], av, gsems[buf]).wait()

        @pl.when(ci >= 2)
        def _():
            pltpu.make_async_copy(mv, s_sh.at[scidxs[buf]],
                                  ssems[buf]).wait()

        def body(i, c2):
            for j in range(D // 16):
                sl = pl.ds(j * 16, 16)
                mv[i, sl] = jnp.maximum(av[i, sl], 0.0)
            return c2

        lax.fori_loop(0, CS, body, 0)
        for k in range(CS // 16):
            sl = pl.ds(k * 16, 16)
            scidxs[buf][sl] = cidxs[buf][sl]
        pltpu.async_copy(mv, s_sh.at[scidxs[buf]], ssems[buf], add=True)

    fire(0, 0)
    fire(1, 1)
    gfire(0, 0)

    def pair(k, carry):
        ci = k * 2
        gfire(ci + 1, 1)
        consume(ci, 0)

        @pl.when(ci + 2 < NCHS)
        def _():
            fire(ci + 2, 0)

        consume(ci + 1, 1)

        @pl.when(ci + 3 < NCHS)
        def _():
            fire(ci + 3, 1)

        @pl.when(ci + 2 < NCHS)
        def _():
            gfire(ci + 2, 0)

        return carry

    lax.fori_loop(0, NCHS // 2, pair, 0)
    if NCHS % 2:
        consume(jnp.int32(NCHS - 1), 0)
    pltpu.make_async_copy(mvs[0], s_sh.at[scidxs[0]], ssems[0]).wait()
    pltpu.make_async_copy(mvs[1], s_sh.at[scidxs[1]], ssems[1]).wait()
    plsc.subcore_barrier()

    sl = pl.ds(sid * RPT, RPT)
    pltpu.sync_copy(s_sh.at[sl], s_out.at[cid, sl])


def kernel(x, pos, edge_index, Wp1, bp1, Wp2, bp2, Wm1, bm1, Wm2, bm2,
           Wu1, bu1, Wu2, bu2):
    row = edge_index[0]
    col = edge_index[1]
    pos8 = jnp.pad(pos, ((0, 0), (0, 5)))
    wp18 = jnp.pad(Wp1, ((0, 5), (0, 0)))
    W2c = Wp2 @ Wm1[2 * D:]
    b2c = (bp2 @ Wm1[2 * D:] + bm1).reshape(1, D)
    W2c4 = jnp.zeros((D, 4 * D), _f32)
    for k in range(4):
        W2c4 = W2c4.at[k * DQ:(k + 1) * DQ, k * D:(k + 1) * D].set(W2c)
    b2c4 = jnp.tile(b2c, (1, 4))

    RB = 1000
    nb = N // RB
    k1 = pl.pallas_call(
        _k1_body,
        grid=(nb,),
        in_specs=[
            pl.BlockSpec((RB, D), lambda i: (i, 0)),
            pl.BlockSpec((RB, 8), lambda i: (i, 0)),
            pl.BlockSpec((D, D), lambda i: (0, 0)),
            pl.BlockSpec((D, D), lambda i: (0, 0)),
            pl.BlockSpec((D, D), lambda i: (0, 0)),
            pl.BlockSpec((8, DQ), lambda i: (0, 0)),
        ],
        out_specs=[
            pl.BlockSpec((RB, D), lambda i: (i, 0)),
            pl.BlockSpec((RB, D), lambda i: (i, 0)),
            pl.BlockSpec((RB, D), lambda i: (i, 0)),
            pl.BlockSpec((RB, DQ), lambda i: (i, 0)),
        ],
        out_shape=[
            jax.ShapeDtypeStruct((N, D), _f32),
            jax.ShapeDtypeStruct((N, D), _f32),
            jax.ShapeDtypeStruct((N, D), _f32),
            jax.ShapeDtypeStruct((N, DQ), _f32),
        ],
    )
    A, B, X1, P = k1(x, pos8, Wm1[:D], Wm1[D:2 * D], Wu1[:D], wp18)

    h, Dg = _sc_h(P, row, col, bp1)

    RE4 = 1000
    k2 = pl.pallas_call(
        _k2_body,
        grid=(E // 4 // RE4,),
        in_specs=[
            pl.BlockSpec((RE4, D), lambda i: (i, 0)),
            pl.BlockSpec((D, 4 * D), lambda i: (0, 0)),
            pl.BlockSpec((1, 4 * D), lambda i: (0, 0)),
        ],
        out_specs=pl.BlockSpec((4 * RE4, D), lambda i: (i, 0)),
        out_shape=jax.ShapeDtypeStruct((E, D), _f32),
    )
    p = k2(h, W2c4, b2c4)

    S = _sc_scatter(A, B, p, row, col)

    k4 = pl.pallas_call(
        _k4_body,
        grid=(nb,),
        in_specs=[
            pl.BlockSpec((RB, D), lambda i: (i, 0)),
            pl.BlockSpec((RB, D), lambda i: (i, 0)),
            pl.BlockSpec((RB, D), lambda i: (i, 0)),
            pl.BlockSpec((RB, 16), lambda i: (i, 0)),
            pl.BlockSpec((RB, 16), lambda i: (i, 0)),
            pl.BlockSpec((D, D), lambda i: (0, 0)),
            pl.BlockSpec((1, D), lambda i: (0, 0)),
            pl.BlockSpec((D, D), lambda i: (0, 0)),
            pl.BlockSpec((1, D), lambda i: (0, 0)),
            pl.BlockSpec((D, D), lambda i: (0, 0)),
            pl.BlockSpec((1, D), lambda i: (0, 0)),
        ],
        out_specs=pl.BlockSpec((RB, D), lambda i: (i, 0)),
        out_shape=jax.ShapeDtypeStruct((N, D), _f32),
    )
    out = k4(X1, S[0], S[1], Dg[0], Dg[1], Wm2, bm2.reshape(1, D),
             Wu1[D:], bu1.reshape(1, D), Wu2, bu2.reshape(1, D))
    return out
